# baseline (speedup 1.0000x reference)
import jax
import jax.numpy as jnp
from jax import lax
from jax.experimental import pallas as pl
from jax.experimental.pallas import tpu as pltpu

_NUM_EXPERTS = 64
_TOP_K = 2
_D = 768
_B = 4
_S = 8192

_CH = 128
_NSTREAM = 16
_NK = _S // (_CH * _NSTREAM)


def _fused_body(*refs):
    (x_refs, (w_ref, gain_ref), (comb_ref, idx_ref, top_ref),
     (wc_ref, acc_ref)) = (refs[:_NSTREAM], refs[_NSTREAM:_NSTREAM + 2],
                           refs[_NSTREAM + 2:_NSTREAM + 5],
                           refs[_NSTREAM + 5:])
    b = pl.program_id(0)
    k = pl.program_id(1)

    @pl.when(jnp.logical_and(b == 0, k == 0))
    def _():
        w = w_ref[...]
        wc_ref[...] = w - jnp.mean(w, axis=1, keepdims=True)

    @pl.when(k == 0)
    def _():
        acc_ref[...] = jnp.zeros_like(acc_ref)

    total = jnp.zeros((1, _NUM_EXPERTS), jnp.float32)
    for x_ref in x_refs:
        chunk = lax.dot_general(
            x_ref[0], wc_ref[...], (((1,), (1,)), ((), ())),
            preferred_element_type=jnp.float32)
        total = total + jnp.sum(chunk, axis=0, keepdims=True)
    acc_ref[...] += total

    @pl.when(k == _NK - 1)
    def _():
        s = acc_ref[...] * gain_ref[...] * (1.0 / _S)
        m = jnp.max(s, axis=1, keepdims=True)
        e = jnp.exp(s - m)
        p = e / jnp.sum(e, axis=1, keepdims=True)
        eidx = lax.broadcasted_iota(jnp.int32, p.shape, 1)
        big = jnp.int32(2 ** 30)
        p1 = jnp.max(p, axis=1, keepdims=True)
        i1 = jnp.min(jnp.where(p == p1, eidx, big), axis=1, keepdims=True)
        pm = jnp.where(eidx == i1, -jnp.inf, p)
        p2 = jnp.max(pm, axis=1, keepdims=True)
        i2 = jnp.min(jnp.where(pm == p2, eidx, big), axis=1, keepdims=True)
        den = p1 + p2 + 1e-9
        comb = (jnp.where(eidx == i1, p1 / den, 0.0)
                + jnp.where(eidx == i2, p2 / den, 0.0))
        rowe = lax.broadcasted_iota(jnp.int32, (_B, _NUM_EXPERTS), 0)
        comb_ref[...] = jnp.where(
            rowe == b, jnp.broadcast_to(comb, (_B, _NUM_EXPERTS)),
            comb_ref[...])
        rankk = lax.broadcasted_iota(jnp.int32, (_TOP_K, _B), 0)
        colb = lax.broadcasted_iota(jnp.int32, (_TOP_K, _B), 1)
        idxv = jnp.where(rankk == 0,
                         jnp.broadcast_to(i1, (_TOP_K, _B)),
                         jnp.broadcast_to(i2, (_TOP_K, _B)))
        topv = jnp.where(rankk == 0,
                         jnp.broadcast_to(p1, (_TOP_K, _B)),
                         jnp.broadcast_to(p2, (_TOP_K, _B)))
        idx_ref[...] = jnp.where(colb == b, idxv, idx_ref[...])
        top_ref[...] = jnp.where(colb == b, topv, top_ref[...])


_fused = pl.pallas_call(
    _fused_body,
    grid=(_B, _NK),
    in_specs=[
        *[pl.BlockSpec((1, _CH, _D),
                       (lambda j: lambda b, k: (b, _NSTREAM * k + j, 0))(j))
          for j in range(_NSTREAM)],
        pl.BlockSpec((_NUM_EXPERTS, _D), lambda b, k: (0, 0)),
        pl.BlockSpec((1, _NUM_EXPERTS), lambda b, k: (0, 0)),
    ],
    out_specs=(
        pl.BlockSpec((_B, _NUM_EXPERTS), lambda b, k: (0, 0)),
        pl.BlockSpec((_TOP_K, _B), lambda b, k: (0, 0)),
        pl.BlockSpec((_TOP_K, _B), lambda b, k: (0, 0)),
    ),
    out_shape=(
        jax.ShapeDtypeStruct((_B, _NUM_EXPERTS), jnp.float32),
        jax.ShapeDtypeStruct((_TOP_K, _B), jnp.int32),
        jax.ShapeDtypeStruct((_TOP_K, _B), jnp.float32),
    ),
    scratch_shapes=[
        pltpu.VMEM((_NUM_EXPERTS, _D), jnp.float32),
        pltpu.VMEM((1, _NUM_EXPERTS), jnp.float32),
    ],
)


def kernel(x, W, gain):
    comb, idx_t, top_t = _fused(*([x] * _NSTREAM), W,
                                gain.reshape(1, _NUM_EXPERTS))
    return comb, idx_t.T, top_t.T


# speedup vs baseline: 1.0860x; 1.0860x over previous
import jax
import jax.numpy as jnp
from jax import lax
from jax.experimental import pallas as pl
from jax.experimental.pallas import tpu as pltpu

_NUM_EXPERTS = 64
_TOP_K = 2
_D = 768
_B = 4
_S = 8192

_CH = 256
_NSTREAM = 16
_NK = _S // (_CH * _NSTREAM)


def _fused_body(*refs):
    (x_refs, (w_ref, gain_ref), (comb_ref, idx_ref, top_ref),
     (wc_ref, acc_ref)) = (refs[:_NSTREAM], refs[_NSTREAM:_NSTREAM + 2],
                           refs[_NSTREAM + 2:_NSTREAM + 5],
                           refs[_NSTREAM + 5:])
    b = pl.program_id(0)
    k = pl.program_id(1)

    @pl.when(jnp.logical_and(b == 0, k == 0))
    def _():
        w = w_ref[...]
        wc_ref[...] = w - jnp.mean(w, axis=1, keepdims=True)

    @pl.when(k == 0)
    def _():
        acc_ref[...] = jnp.zeros_like(acc_ref)

    total = jnp.zeros((1, _NUM_EXPERTS), jnp.float32)
    for x_ref in x_refs:
        chunk = lax.dot_general(
            x_ref[0], wc_ref[...], (((1,), (1,)), ((), ())),
            preferred_element_type=jnp.float32)
        total = total + jnp.sum(chunk, axis=0, keepdims=True)
    acc_ref[...] += total

    @pl.when(k == _NK - 1)
    def _():
        s = acc_ref[...] * gain_ref[...] * (1.0 / _S)
        m = jnp.max(s, axis=1, keepdims=True)
        e = jnp.exp(s - m)
        p = e / jnp.sum(e, axis=1, keepdims=True)
        eidx = lax.broadcasted_iota(jnp.int32, p.shape, 1)
        big = jnp.int32(2 ** 30)
        p1 = jnp.max(p, axis=1, keepdims=True)
        i1 = jnp.min(jnp.where(p == p1, eidx, big), axis=1, keepdims=True)
        pm = jnp.where(eidx == i1, -jnp.inf, p)
        p2 = jnp.max(pm, axis=1, keepdims=True)
        i2 = jnp.min(jnp.where(pm == p2, eidx, big), axis=1, keepdims=True)
        den = p1 + p2 + 1e-9
        comb = (jnp.where(eidx == i1, p1 / den, 0.0)
                + jnp.where(eidx == i2, p2 / den, 0.0))
        rowe = lax.broadcasted_iota(jnp.int32, (_B, _NUM_EXPERTS), 0)
        comb_ref[...] = jnp.where(
            rowe == b, jnp.broadcast_to(comb, (_B, _NUM_EXPERTS)),
            comb_ref[...])
        rankk = lax.broadcasted_iota(jnp.int32, (_TOP_K, _B), 0)
        colb = lax.broadcasted_iota(jnp.int32, (_TOP_K, _B), 1)
        idxv = jnp.where(rankk == 0,
                         jnp.broadcast_to(i1, (_TOP_K, _B)),
                         jnp.broadcast_to(i2, (_TOP_K, _B)))
        topv = jnp.where(rankk == 0,
                         jnp.broadcast_to(p1, (_TOP_K, _B)),
                         jnp.broadcast_to(p2, (_TOP_K, _B)))
        idx_ref[...] = jnp.where(colb == b, idxv, idx_ref[...])
        top_ref[...] = jnp.where(colb == b, topv, top_ref[...])


_fused = pl.pallas_call(
    _fused_body,
    grid=(_B, _NK),
    in_specs=[
        *[pl.BlockSpec((1, _CH, _D),
                       (lambda j: lambda b, k: (b, _NSTREAM * k + j, 0))(j))
          for j in range(_NSTREAM)],
        pl.BlockSpec((_NUM_EXPERTS, _D), lambda b, k: (0, 0)),
        pl.BlockSpec((1, _NUM_EXPERTS), lambda b, k: (0, 0)),
    ],
    out_specs=(
        pl.BlockSpec((_B, _NUM_EXPERTS), lambda b, k: (0, 0)),
        pl.BlockSpec((_TOP_K, _B), lambda b, k: (0, 0)),
        pl.BlockSpec((_TOP_K, _B), lambda b, k: (0, 0)),
    ),
    out_shape=(
        jax.ShapeDtypeStruct((_B, _NUM_EXPERTS), jnp.float32),
        jax.ShapeDtypeStruct((_TOP_K, _B), jnp.int32),
        jax.ShapeDtypeStruct((_TOP_K, _B), jnp.float32),
    ),
    scratch_shapes=[
        pltpu.VMEM((_NUM_EXPERTS, _D), jnp.float32),
        pltpu.VMEM((1, _NUM_EXPERTS), jnp.float32),
    ],
)


def kernel(x, W, gain):
    comb, idx_t, top_t = _fused(*([x] * _NSTREAM), W,
                                gain.reshape(1, _NUM_EXPERTS))
    return comb, idx_t.T, top_t.T


# speedup vs baseline: 1.0914x; 1.0050x over previous
import jax
import jax.numpy as jnp
from jax import lax
from jax.experimental import pallas as pl
from jax.experimental.pallas import tpu as pltpu

_NUM_EXPERTS = 64
_TOP_K = 2
_D = 768
_B = 4
_S = 8192

_CH = 256
_NSTREAM = 16
_NK = _S // (_CH * _NSTREAM)


def _fused_body(*refs):
    (x_refs, (w_ref, gain_ref), (comb_ref, idx_ref, top_ref),
     (wc_ref, acc_ref)) = (refs[:_NSTREAM], refs[_NSTREAM:_NSTREAM + 2],
                           refs[_NSTREAM + 2:_NSTREAM + 5],
                           refs[_NSTREAM + 5:])
    b = pl.program_id(0)
    k = pl.program_id(1)

    @pl.when(jnp.logical_and(b == 0, k == 0))
    def _():
        w = w_ref[...]
        wc_ref[...] = w - jnp.mean(w, axis=1, keepdims=True)

    @pl.when(k == 0)
    def _():
        acc_ref[...] = jnp.zeros_like(acc_ref)

    total = jnp.zeros((1, _NUM_EXPERTS), jnp.float32)
    for x_ref in x_refs:
        chunk = lax.dot_general(
            x_ref[0], wc_ref[...], (((1,), (1,)), ((), ())),
            preferred_element_type=jnp.float32)
        total = total + jnp.sum(chunk, axis=0, keepdims=True)
    acc_ref[...] += total

    @pl.when(k == _NK - 1)
    def _():
        s = acc_ref[...] * gain_ref[...] * (1.0 / _S)
        m = jnp.max(s, axis=1, keepdims=True)
        e = jnp.exp(s - m)
        p = e / jnp.sum(e, axis=1, keepdims=True)
        eidx = lax.broadcasted_iota(jnp.int32, p.shape, 1)
        big = jnp.int32(2 ** 30)
        p1 = jnp.max(p, axis=1, keepdims=True)
        i1 = jnp.min(jnp.where(p == p1, eidx, big), axis=1, keepdims=True)
        pm = jnp.where(eidx == i1, -jnp.inf, p)
        p2 = jnp.max(pm, axis=1, keepdims=True)
        i2 = jnp.min(jnp.where(pm == p2, eidx, big), axis=1, keepdims=True)
        den = p1 + p2 + 1e-9
        comb = (jnp.where(eidx == i1, p1 / den, 0.0)
                + jnp.where(eidx == i2, p2 / den, 0.0))
        rowe = lax.broadcasted_iota(jnp.int32, (_B, _NUM_EXPERTS), 0)
        comb_ref[...] = jnp.where(
            rowe == b, jnp.broadcast_to(comb, (_B, _NUM_EXPERTS)),
            comb_ref[...])
        rankk = lax.broadcasted_iota(jnp.int32, (_TOP_K, _B), 0)
        colb = lax.broadcasted_iota(jnp.int32, (_TOP_K, _B), 1)
        idxv = jnp.where(rankk == 0,
                         jnp.broadcast_to(i1, (_TOP_K, _B)),
                         jnp.broadcast_to(i2, (_TOP_K, _B)))
        topv = jnp.where(rankk == 0,
                         jnp.broadcast_to(p1, (_TOP_K, _B)),
                         jnp.broadcast_to(p2, (_TOP_K, _B)))
        idx_ref[...] = jnp.where(colb == b, idxv, idx_ref[...])
        top_ref[...] = jnp.where(colb == b, topv, top_ref[...])


_fused = pl.pallas_call(
    _fused_body,
    grid=(_B, _NK),
    in_specs=[
        *[pl.BlockSpec((1, _CH, _D),
                       (lambda j: lambda b, k: (b, j * _NK + k, 0))(j))
          for j in range(_NSTREAM)],
        pl.BlockSpec((_NUM_EXPERTS, _D), lambda b, k: (0, 0)),
        pl.BlockSpec((1, _NUM_EXPERTS), lambda b, k: (0, 0)),
    ],
    out_specs=(
        pl.BlockSpec((_B, _NUM_EXPERTS), lambda b, k: (0, 0)),
        pl.BlockSpec((_TOP_K, _B), lambda b, k: (0, 0)),
        pl.BlockSpec((_TOP_K, _B), lambda b, k: (0, 0)),
    ),
    out_shape=(
        jax.ShapeDtypeStruct((_B, _NUM_EXPERTS), jnp.float32),
        jax.ShapeDtypeStruct((_TOP_K, _B), jnp.int32),
        jax.ShapeDtypeStruct((_TOP_K, _B), jnp.float32),
    ),
    scratch_shapes=[
        pltpu.VMEM((_NUM_EXPERTS, _D), jnp.float32),
        pltpu.VMEM((1, _NUM_EXPERTS), jnp.float32),
    ],
)


def kernel(x, W, gain):
    comb, idx_t, top_t = _fused(*([x] * _NSTREAM), W,
                                gain.reshape(1, _NUM_EXPERTS))
    return comb, idx_t.T, top_t.T
